# Initial kernel scaffold; baseline (speedup 1.0000x reference)
import jax
import jax.numpy as jnp
from jax.experimental import pallas as pl


def kernel(tokens, tags, word_table, pos_table):
    raise NotImplementedError("write your pallas kernel here")


# speedup vs baseline: 1.1433x; 1.1433x over previous
import functools

import jax
import jax.numpy as jnp
from jax import lax
from jax.experimental import pallas as pl
from jax.experimental.pallas import tpu as pltpu
from jax.experimental.pallas import tpu_sc as plsc

WORD_DIM = 300
POS_DIM = 50
OUT_DIM = WORD_DIM + POS_DIM
WPAD = 384
PPAD = 128
NUM_CORES = 2
NUM_SUBCORES = 16
NUM_WORKERS = NUM_CORES * NUM_SUBCORES
CHUNK = 128


def kernel(tokens, tags, word_table, pos_table):
    B, L = tokens.shape
    N = B * L
    n_per_w = N // NUM_WORKERS
    n_chunks = n_per_w // CHUNK

    tok = tokens.reshape(N).astype(jnp.int32)
    tag = tags.reshape(N).astype(jnp.int32)
    wpad = jnp.pad(word_table, ((0, 0), (0, WPAD - WORD_DIM)))
    ppad = jnp.pad(pos_table, ((0, 0), (0, PPAD - POS_DIM)))

    mesh = plsc.VectorSubcoreMesh(core_axis_name="c", subcore_axis_name="s")

    @functools.partial(
        pl.kernel,
        mesh=mesh,
        out_type=(
            jax.ShapeDtypeStruct((N, WPAD), jnp.float32),
            jax.ShapeDtypeStruct((N, PPAD), jnp.float32),
        ),
        scratch_types=[
            pltpu.VMEM((CHUNK,), jnp.int32),
            pltpu.VMEM((CHUNK,), jnp.int32),
            pltpu.VMEM((CHUNK, WPAD), jnp.float32),
            pltpu.VMEM((CHUNK, PPAD), jnp.float32),
            pltpu.SemaphoreType.DMA,
            pltpu.SemaphoreType.DMA,
        ],
    )
    def gather_sc(tok_hbm, tag_hbm, word_hbm, pos_hbm, wout_hbm, pout_hbm,
                  tok_v, tag_v, word_v, pos_v, sem_w, sem_p):
        wid = lax.axis_index("s") * NUM_CORES + lax.axis_index("c")

        @pl.loop(0, n_chunks)
        def _(i):
            base = wid * n_per_w + i * CHUNK
            pltpu.sync_copy(tok_hbm.at[pl.ds(base, CHUNK)], tok_v)
            pltpu.sync_copy(tag_hbm.at[pl.ds(base, CHUNK)], tag_v)
            cw = pltpu.async_copy(word_hbm.at[tok_v], word_v, sem_w)
            cp = pltpu.async_copy(pos_hbm.at[tag_v], pos_v, sem_p)
            cw.wait()
            cp.wait()
            pltpu.sync_copy(word_v, wout_hbm.at[pl.ds(base, CHUNK)])
            pltpu.sync_copy(pos_v, pout_hbm.at[pl.ds(base, CHUNK)])

    wout, pout = gather_sc(tok, tag, wpad, ppad)
    out = jnp.concatenate(
        [wout[:, :WORD_DIM], pout[:, :POS_DIM]], axis=1)
    return out.reshape(B, L, OUT_DIM)


# speedup vs baseline: 1.6842x; 1.4731x over previous
import functools

import jax
import jax.numpy as jnp
from jax import lax
from jax.experimental import pallas as pl
from jax.experimental.pallas import tpu as pltpu
from jax.experimental.pallas import tpu_sc as plsc

WORD_DIM = 300
POS_DIM = 50
OUT_DIM = WORD_DIM + POS_DIM
WPAD = 384
TILE = 128
PSHIFT = WORD_DIM - 2 * TILE
NUM_CORES = 2
NUM_SUBCORES = 16
NUM_WORKERS = NUM_CORES * NUM_SUBCORES
CHUNK = 128


def kernel(tokens, tags, word_table, pos_table):
    B, L = tokens.shape
    N = B * L
    n_per_w = N // NUM_WORKERS
    n_chunks = n_per_w // CHUNK

    tok = tokens.reshape(N).astype(jnp.int32)
    tag = tags.reshape(N).astype(jnp.int32)
    wt0 = word_table[:, :TILE]
    wt1 = word_table[:, TILE:2 * TILE]
    wt2 = jnp.pad(word_table[:, 2 * TILE:], ((0, 0), (0, WPAD - WORD_DIM)))
    ppad = jnp.pad(pos_table,
                   ((0, 0), (PSHIFT, TILE - POS_DIM - PSHIFT)))

    mesh = plsc.VectorSubcoreMesh(core_axis_name="c", subcore_axis_name="s")

    @functools.partial(
        pl.kernel,
        mesh=mesh,
        out_type=jax.ShapeDtypeStruct((N, WPAD), jnp.float32),
        scratch_types=[
            pltpu.VMEM((CHUNK,), jnp.int32),
            pltpu.VMEM((CHUNK,), jnp.int32),
            pltpu.VMEM((CHUNK, TILE), jnp.float32),
            pltpu.VMEM((CHUNK, TILE), jnp.float32),
            pltpu.VMEM((CHUNK, TILE), jnp.float32),
            pltpu.VMEM((CHUNK, TILE), jnp.float32),
            pltpu.SemaphoreType.DMA,
            pltpu.SemaphoreType.DMA,
            pltpu.SemaphoreType.DMA,
            pltpu.SemaphoreType.DMA,
        ],
    )
    def gather_sc(tok_hbm, tag_hbm, w0_hbm, w1_hbm, w2_hbm, pos_hbm, out_hbm,
                  tok_v, tag_v, v0, v1, v2, pv, s0, s1, s2, sp):
        wid = lax.axis_index("s") * NUM_CORES + lax.axis_index("c")
        wbase = wid * n_per_w

        @pl.loop(0, n_chunks)
        def _(i):
            base = wbase + i * CHUNK
            pltpu.sync_copy(tok_hbm.at[pl.ds(base, CHUNK)], tok_v)
            pltpu.sync_copy(tag_hbm.at[pl.ds(base, CHUNK)], tag_v)
            c0 = pltpu.async_copy(w0_hbm.at[tok_v], v0, s0)
            c1 = pltpu.async_copy(w1_hbm.at[tok_v], v1, s1)
            c2 = pltpu.async_copy(w2_hbm.at[tok_v], v2, s2)
            cp = pltpu.async_copy(pos_hbm.at[tag_v], pv, sp)
            c0.wait()
            c1.wait()
            c2.wait()
            cp.wait()

            @pl.loop(0, CHUNK)
            def _(r):
                for k in range(TILE // 16):
                    v2[r, pl.ds(16 * k, 16)] = (
                        v2[r, pl.ds(16 * k, 16)] + pv[r, pl.ds(16 * k, 16)])

            pltpu.sync_copy(
                v0, out_hbm.at[pl.ds(base, CHUNK), pl.ds(0, TILE)])
            pltpu.sync_copy(
                v1, out_hbm.at[pl.ds(base, CHUNK), pl.ds(TILE, TILE)])
            pltpu.sync_copy(
                v2, out_hbm.at[pl.ds(base, CHUNK), pl.ds(2 * TILE, TILE)])

    out = gather_sc(tok, tag, wt0, wt1, wt2, ppad)
    return out[:, :OUT_DIM].reshape(B, L, OUT_DIM)
